# baseline (speedup 1.0000x reference)
import functools

import jax
import jax.numpy as jnp
import numpy as np
from jax import lax
from jax.experimental import pallas as pl
from jax.experimental.pallas import tpu as pltpu
from jax.experimental.pallas import tpu_sc as plsc

_TOKENS = 512
_N, _L, _D = 128, 1025, 96
_T1 = _TOKENS + 1
_CHUNK = 128
_SLOTS = 4
_NW = 32
_CPW = 17

_ROT_A = (13, 15, 26, 6)
_ROT_B = (17, 29, 16, 24)


def _rotl(x, d):
    return ((x << np.uint32(d)) | (x >> np.uint32(32 - d))).astype(np.uint32)


def _threefry2x32(k0, k1, x0, x1):
    ks0, ks1 = np.uint32(k0), np.uint32(k1)
    ks2 = np.uint32(ks0 ^ ks1 ^ np.uint32(0x1BD11BDA))
    x0 = (x0 + ks0).astype(np.uint32)
    x1 = (x1 + ks1).astype(np.uint32)
    keys = (ks0, ks1, ks2)
    for i, rset in enumerate((_ROT_A, _ROT_B, _ROT_A, _ROT_B, _ROT_A)):
        for r in rset:
            x0 = (x0 + x1).astype(np.uint32)
            x1 = _rotl(x1, r)
            x1 = (x1 ^ x0).astype(np.uint32)
        x0 = (x0 + keys[(i + 1) % 3]).astype(np.uint32)
        x1 = (x1 + keys[(i + 2) % 3] + np.uint32(i + 1)).astype(np.uint32)
    return x0, x1


def _uniform(seed: int, shape) -> np.ndarray:
    n = int(np.prod(shape))
    idx = np.arange(n, dtype=np.uint64)
    b1, b2 = _threefry2x32(
        np.uint32(seed >> 32), np.uint32(seed & 0xFFFFFFFF),
        (idx >> np.uint64(32)).astype(np.uint32),
        (idx & np.uint64(0xFFFFFFFF)).astype(np.uint32),
    )
    bits = (b1 ^ b2).astype(np.uint32)
    fl = ((bits >> np.uint32(9)) | np.uint32(0x3F800000)).view(np.float32)
    return np.maximum(np.float32(0.0), fl - np.float32(1.0)).reshape(shape)


@functools.lru_cache(maxsize=1)
def _worker_indices() -> np.ndarray:
    scores = _uniform(42, (_N, _L - 1))
    patch = np.argsort(scores, axis=1, kind="stable")[:, :_TOKENS] + 1
    patch = np.sort(patch, axis=1).astype(np.int32)
    mask = np.concatenate(
        [np.zeros((_N, 1), np.int32), patch], axis=1)
    nvec = np.arange(_N, dtype=np.int32)
    src = mask.T * _N + nvec[None, :]
    idxw = np.zeros((_NW, _CPW, _CHUNK), np.int32)
    for w in range(_NW):
        for c in range(_CPW):
            tp = w + _NW * c
            if tp < _T1:
                idxw[w, c] = src[tp]
    return np.ascontiguousarray(idxw.reshape(_NW, _CPW * _CHUNK))


@functools.lru_cache(maxsize=1)
def _sc_gather():
    info = plsc.get_sparse_core_info()
    nc = info.num_cores
    mesh = plsc.VectorSubcoreMesh(core_axis_name="c", subcore_axis_name="s")

    @functools.partial(
        pl.kernel,
        mesh=mesh,
        out_type=jax.ShapeDtypeStruct((_T1 * _N, _D), jnp.float32),
        scratch_types=[
            pltpu.VMEM((_CPW * _CHUNK,), jnp.int32),
            pltpu.VMEM((_SLOTS, _CHUNK, _D), jnp.float32),
            [pltpu.SemaphoreType.DMA] * _SLOTS,
            [pltpu.SemaphoreType.DMA] * _SLOTS,
        ],
        compiler_params=pltpu.CompilerParams(use_tc_tiling_on_sc=False),
    )
    def gather_kernel(xf, idxf, out, idx_v, gbuf, gsems, wsems):
        wid = lax.axis_index("s") * nc + lax.axis_index("c")
        pltpu.sync_copy(idxf.at[wid], idx_v)

        gather_pend = {}
        write_pend = {}

        def issue(c):
            slot = c % _SLOTS
            if c >= _SLOTS:
                write_pend.pop(c - _SLOTS).wait()
            gather_pend[c] = pltpu.async_copy(
                xf.at[idx_v.at[pl.ds(c * _CHUNK, _CHUNK)]],
                gbuf.at[slot], gsems[slot])

        def retire(c):
            slot = c % _SLOTS
            tp = wid + _NW * c
            gather_pend.pop(c).wait()
            write_pend[c] = pltpu.async_copy(
                gbuf.at[slot], out.at[pl.ds(tp * _CHUNK, _CHUNK)], wsems[slot])

        nfull = _CPW - 1
        for c in range(min(_SLOTS - 1, nfull)):
            issue(c)
        for c in range(nfull):
            if c + _SLOTS - 1 < nfull:
                issue(c + _SLOTS - 1)
            retire(c)
        for c in sorted(write_pend):
            write_pend.pop(c).wait()

        @pl.when(wid == 0)
        def _():
            pltpu.async_copy(
                xf.at[idx_v.at[pl.ds(nfull * _CHUNK, _CHUNK)]],
                gbuf.at[0], gsems[0]).wait()
            pltpu.sync_copy(
                gbuf.at[0], out.at[pl.ds(_TOKENS * _CHUNK, _CHUNK)])

    return gather_kernel


def kernel(x):
    n, l, d = x.shape
    xq = jnp.transpose(x, (1, 0, 2)).reshape(l * n, d)
    idxf = jnp.asarray(_worker_indices())
    o = _sc_gather()(xq, idxf)
    return jnp.transpose(o.reshape(_T1, n, d), (1, 0, 2))
